# Initial kernel scaffold; baseline (speedup 1.0000x reference)
import jax
import jax.numpy as jnp
from jax.experimental import pallas as pl


def kernel(embeddings, target):
    raise NotImplementedError("write your pallas kernel here")


# speedup vs baseline: 1.2944x; 1.2944x over previous
import jax
import jax.numpy as jnp
from jax import lax
from jax.experimental import pallas as pl

MARGIN = 1.0
EPS = 1e-6

BM = 512
BN = 512


def _loss_kernel(trow_ref, tcol_ref, a_ref, b_ref, out_ref):
    i = pl.program_id(0)
    j = pl.program_id(1)

    @pl.when((i == 0) & (j == 0))
    def _init():
        out_ref[...] = jnp.zeros_like(out_ref)

    @pl.when(j >= i)
    def _body():
        a = a_ref[...]
        b = b_ref[...]
        d_feat = a.shape[1]
        g = lax.dot_general(a, b, (((1,), (1,)), ((), ())),
                            preferred_element_type=jnp.float32)
        sqa = jnp.sum(a * a, axis=1, keepdims=True)
        sa = jnp.sum(a, axis=1, keepdims=True)
        sqb = jnp.sum(b * b, axis=1)[None, :]
        sb = jnp.sum(b, axis=1)[None, :]
        d2 = sqa + sqb - 2.0 * g + (2.0 * EPS) * (sa - sb) + d_feat * EPS * EPS
        d2 = jnp.maximum(d2, 0.0)
        d = jnp.sqrt(d2)
        same = trow_ref[...] == tcol_ref[...]
        rows = i * BM + lax.broadcasted_iota(jnp.int32, (BM, BN), 0)
        cols = j * BN + lax.broadcasted_iota(jnp.int32, (BM, BN), 1)
        triu = cols > rows
        pos = jnp.where(same & triu, d2, 0.0)
        h = jnp.maximum(MARGIN - d, 0.0)
        neg = jnp.where((~same) & triu, h * h, 0.0)
        out_ref[...] += jnp.sum(pos + neg, keepdims=True)


@jax.jit
def kernel(embeddings, target):
    B, D = embeddings.shape
    trow = target.reshape(B, 1)
    tcol = target.reshape(1, B)
    grid = (B // BM, B // BN)
    out = pl.pallas_call(
        _loss_kernel,
        grid=grid,
        in_specs=[
            pl.BlockSpec((BM, 1), lambda i, j: (i, 0)),
            pl.BlockSpec((1, BN), lambda i, j: (0, j)),
            pl.BlockSpec((BM, D), lambda i, j: (i, 0)),
            pl.BlockSpec((BN, D), lambda i, j: (j, 0)),
        ],
        out_specs=pl.BlockSpec((1, 1), lambda i, j: (0, 0)),
        out_shape=jax.ShapeDtypeStruct((1, 1), jnp.float32),
    )(trow, tcol, embeddings, embeddings)
    return out[0, 0]


# speedup vs baseline: 1.8500x; 1.4292x over previous
import jax
import jax.numpy as jnp
from jax import lax
from jax.experimental import pallas as pl
from jax.experimental.pallas import tpu as pltpu

MARGIN = 1.0
EPS = 1e-6

BM = 512
BN = 512


def _loss_kernel(im_ref, jm_ref, trow_ref, tcol_ref, a_ref, b_ref, out_ref,
                 acc_ref):
    k = pl.program_id(0)
    nsteps = pl.num_programs(0)
    i = im_ref[k]
    j = jm_ref[k]

    @pl.when(k == 0)
    def _init():
        acc_ref[...] = jnp.zeros_like(acc_ref)

    a = a_ref[...]
    b = b_ref[...]
    d_feat = a.shape[1]
    g = lax.dot_general(a, b, (((1,), (1,)), ((), ())),
                        preferred_element_type=jnp.float32)
    ra = (jnp.sum(a * a, axis=1, keepdims=True)
          + (2.0 * EPS) * jnp.sum(a, axis=1, keepdims=True)
          + d_feat * EPS * EPS)
    cb = (jnp.sum(b * b, axis=1)
          - (2.0 * EPS) * jnp.sum(b, axis=1))[None, :]
    d2 = jnp.maximum(ra + cb - 2.0 * g, 0.0)
    d = jnp.sqrt(d2)
    h = jnp.maximum(MARGIN - d, 0.0)
    same = trow_ref[...] == tcol_ref[...]
    val = jnp.where(same, d2, h * h)

    @pl.when(i == j)
    def _diag():
        rows = lax.broadcasted_iota(jnp.int32, (BM, BN), 0)
        cols = lax.broadcasted_iota(jnp.int32, (BM, BN), 1)
        acc_ref[...] += jnp.sum(jnp.where(cols > rows, val, 0.0),
                                axis=0, keepdims=True)

    @pl.when(i != j)
    def _offdiag():
        acc_ref[...] += jnp.sum(val, axis=0, keepdims=True)

    @pl.when(k == nsteps - 1)
    def _fini():
        out_ref[...] = jnp.sum(acc_ref[...], axis=1, keepdims=True)


@jax.jit
def kernel(embeddings, target):
    B, D = embeddings.shape
    nbm, nbn = B // BM, B // BN
    pairs = [(i, j) for i in range(nbm) for j in range(i, nbn)]
    im = jnp.asarray([p[0] for p in pairs], dtype=jnp.int32)
    jm = jnp.asarray([p[1] for p in pairs], dtype=jnp.int32)
    trow = target.reshape(B, 1)
    tcol = target.reshape(1, B)
    grid_spec = pltpu.PrefetchScalarGridSpec(
        num_scalar_prefetch=2,
        grid=(len(pairs),),
        in_specs=[
            pl.BlockSpec((BM, 1), lambda k, im, jm: (im[k], 0)),
            pl.BlockSpec((1, BN), lambda k, im, jm: (0, jm[k])),
            pl.BlockSpec((BM, D), lambda k, im, jm: (im[k], 0)),
            pl.BlockSpec((BN, D), lambda k, im, jm: (jm[k], 0)),
        ],
        out_specs=pl.BlockSpec((1, 1), lambda k, im, jm: (0, 0)),
        scratch_shapes=[pltpu.VMEM((1, BN), jnp.float32)],
    )
    out = pl.pallas_call(
        _loss_kernel,
        grid_spec=grid_spec,
        out_shape=jax.ShapeDtypeStruct((1, 1), jnp.float32),
    )(im, jm, trow, tcol, embeddings, embeddings)
    return out[0, 0]


# speedup vs baseline: 3.0413x; 1.6439x over previous
import jax
import jax.numpy as jnp
from jax import lax
from jax.experimental import pallas as pl
from jax.experimental.pallas import tpu as pltpu

MARGIN = 1.0
EPS = 1e-6

BM = 1024
BN = 1024


def _aug_kernel(e_ref, aug_a_ref, aug_b_ref):
    e = e_ref[...]
    d_feat = e.shape[1]
    sq = jnp.sum(e * e, axis=1, keepdims=True)
    s = jnp.sum(e, axis=1, keepdims=True)
    ra = sq + (2.0 * EPS) * s + d_feat * EPS * EPS
    cb = sq - (2.0 * EPS) * s
    lane = lax.broadcasted_iota(jnp.int32, e.shape, 1)
    extra_a = jnp.where(lane == 0, 1.0, jnp.where(lane == 1, ra, 0.0))
    extra_b = jnp.where(lane == 0, cb, jnp.where(lane == 1, 1.0, 0.0))
    aug_a_ref[...] = jnp.concatenate([e, extra_a], axis=1)
    aug_b_ref[...] = jnp.concatenate([-2.0 * e, extra_b], axis=1)


def _loss_kernel(im_ref, jm_ref, trow_ref, tcol_ref, a_ref, b_ref, out_ref,
                 acc_ref):
    k = pl.program_id(0)
    nsteps = pl.num_programs(0)
    i = im_ref[k]
    j = jm_ref[k]

    @pl.when(k == 0)
    def _init():
        acc_ref[...] = jnp.zeros_like(acc_ref)

    d2 = lax.dot_general(a_ref[...], b_ref[...], (((1,), (1,)), ((), ())),
                         preferred_element_type=jnp.float32)
    d2 = jnp.maximum(d2, 0.0)
    d = d2 * lax.rsqrt(d2 + 1e-30)
    h = jnp.maximum(MARGIN - d, 0.0)
    same = trow_ref[...] == tcol_ref[...]
    val = jnp.where(same, d2, h * h)

    def _accumulate(v):
        total = None
        for c in range(0, BM // 8, 8):
            parts = [v[8 * m:8 * (m + 1), :] for m in range(c, c + 8)]
            while len(parts) > 1:
                parts = [parts[p] + parts[p + 1]
                         for p in range(0, len(parts), 2)]
            total = parts[0] if total is None else total + parts[0]
        acc_ref[...] += total

    @pl.when(i == j)
    def _diag():
        rows = lax.broadcasted_iota(jnp.int32, (BM, BN), 0)
        cols = lax.broadcasted_iota(jnp.int32, (BM, BN), 1)
        _accumulate(jnp.where(cols > rows, val, 0.0))

    @pl.when(i != j)
    def _offdiag():
        _accumulate(val)

    @pl.when(k == nsteps - 1)
    def _fini():
        out_ref[...] = jnp.sum(acc_ref[...], axis=(0, 1), keepdims=True)


@jax.jit
def kernel(embeddings, target):
    B, D = embeddings.shape
    DA = 2 * D
    aug_a, aug_b = pl.pallas_call(
        _aug_kernel,
        out_shape=[jax.ShapeDtypeStruct((B, DA), jnp.float32),
                   jax.ShapeDtypeStruct((B, DA), jnp.float32)],
    )(embeddings)

    nbm, nbn = B // BM, B // BN
    pairs = [(i, j) for i in range(nbm) for j in range(i, nbn)]
    im = jnp.asarray([p[0] for p in pairs], dtype=jnp.int32)
    jm = jnp.asarray([p[1] for p in pairs], dtype=jnp.int32)
    trow = target.reshape(B, 1)
    tcol = target.reshape(1, B)
    grid_spec = pltpu.PrefetchScalarGridSpec(
        num_scalar_prefetch=2,
        grid=(len(pairs),),
        in_specs=[
            pl.BlockSpec((BM, 1), lambda k, im, jm: (im[k], 0)),
            pl.BlockSpec((1, BN), lambda k, im, jm: (0, jm[k])),
            pl.BlockSpec((BM, DA), lambda k, im, jm: (im[k], 0)),
            pl.BlockSpec((BN, DA), lambda k, im, jm: (jm[k], 0)),
        ],
        out_specs=pl.BlockSpec((1, 1), lambda k, im, jm: (0, 0)),
        scratch_shapes=[pltpu.VMEM((8, BN), jnp.float32)],
    )
    out = pl.pallas_call(
        _loss_kernel,
        grid_spec=grid_spec,
        out_shape=jax.ShapeDtypeStruct((1, 1), jnp.float32),
    )(im, jm, trow, tcol, aug_a, aug_b)
    return out[0, 0]


# speedup vs baseline: 4.8599x; 1.5980x over previous
import jax
import jax.numpy as jnp
from jax import lax
from jax.experimental import pallas as pl

MARGIN = 1.0
EPS = 1e-6

BM = 1024
BN = 1024


def _sum_rows(v):
    total = None
    for c in range(0, v.shape[0] // 8, 8):
        parts = [v[8 * m:8 * (m + 1), :] for m in range(c, c + 8)]
        while len(parts) > 1:
            parts = [parts[p] + parts[p + 1] for p in range(0, len(parts), 2)]
        total = parts[0] if total is None else total + parts[0]
    return total


def _loss_kernel(e_ref, trow_ref, tcol_ref, out_ref):
    e = e_ref[...]
    b_rows, d_feat = e.shape
    sq = jnp.sum(e * e, axis=1, keepdims=True)
    s = jnp.sum(e, axis=1, keepdims=True)
    ra = sq + (2.0 * EPS) * s + d_feat * EPS * EPS
    cb = sq - (2.0 * EPS) * s
    lane = lax.broadcasted_iota(jnp.int32, e.shape, 1)
    extra_a = jnp.where(lane == 0, 1.0, jnp.where(lane == 1, ra, 0.0))
    extra_b = jnp.where(lane == 0, cb, jnp.where(lane == 1, 1.0, 0.0))
    aug_a = jnp.concatenate([e, extra_a], axis=1)
    aug_b = jnp.concatenate([-2.0 * e, extra_b], axis=1)

    acc = None
    for i in range(b_rows // BM):
        for j in range(i, b_rows // BN):
            a = aug_a[i * BM:(i + 1) * BM, :]
            b = aug_b[j * BN:(j + 1) * BN, :]
            d2 = lax.dot_general(a, b, (((1,), (1,)), ((), ())),
                                 preferred_element_type=jnp.float32)
            x = jnp.maximum(d2, 1e-30)
            d = x * lax.rsqrt(x)
            h = jnp.maximum(MARGIN - d, 0.0)
            same = (trow_ref[i * BM:(i + 1) * BM, :]
                    == tcol_ref[:, j * BN:(j + 1) * BN])
            val = jnp.where(same, x, h * h)
            if i == j:
                rows = lax.broadcasted_iota(jnp.int32, (BM, BN), 0)
                cols = lax.broadcasted_iota(jnp.int32, (BM, BN), 1)
                val = jnp.where(cols > rows, val, 0.0)
            part = _sum_rows(val)
            acc = part if acc is None else acc + part
    out_ref[...] = jnp.sum(acc, axis=(0, 1), keepdims=True)


@jax.jit
def kernel(embeddings, target):
    B, D = embeddings.shape
    trow = target.reshape(B, 1)
    tcol = target.reshape(1, B)
    out = pl.pallas_call(
        _loss_kernel,
        out_shape=jax.ShapeDtypeStruct((1, 1), jnp.float32),
    )(embeddings, trow, tcol)
    return out[0, 0]
